# baseline (speedup 1.0000x reference)
import jax
import jax.numpy as jnp
from jax.experimental import pallas as pl
from jax.experimental.pallas import tpu as pltpu

D = 768
S = 8192
H = 10
BLOCK_S = 4096


def _b16(x):
    return x.astype(jnp.bfloat16)


def _body(mem_ref, q2d_ref, wq0_ref, wq1_ref, bq0_ref, bq1_ref,
          wk0_ref, wk1_ref, wvt_ref, selt_ref,
          vals_ref, bs_ref, bi_ref,
          q0_s, q1_s, m_s, idx_s, rows_s):
    step = pl.program_id(0)
    nsteps = pl.num_programs(0)

    @pl.when(step == 0)
    def _init():
        q0 = jax.lax.dot_general(
            _b16(q2d_ref[:]), _b16(wq0_ref[:]), (((1,), (1,)), ((), ())),
            preferred_element_type=jnp.float32) + bq0_ref[:]
        q1 = jax.lax.dot_general(
            _b16(q2d_ref[:]), _b16(wq1_ref[:]), (((1,), (1,)), ((), ())),
            preferred_element_type=jnp.float32) + bq1_ref[:]
        q0_s[:] = q0
        q1_s[:] = q1
        m_s[:] = jnp.full((1, H), -jnp.inf, dtype=jnp.float32)
        idx_s[:] = jnp.zeros((1, H), dtype=jnp.int32)
        rows_s[:] = jnp.zeros((D, H), dtype=jnp.float32)

    memb = _b16(mem_ref[:])
    s0 = jax.lax.dot_general(memb, _b16(wk0_ref[:]), (((1,), (1,)), ((), ())),
                             preferred_element_type=jnp.float32)
    s1 = jax.lax.dot_general(memb, _b16(wk1_ref[:]), (((1,), (1,)), ((), ())),
                             preferred_element_type=jnp.float32)
    scores = (_b16(s0).astype(jnp.float32) * _b16(q0_s[:]).astype(jnp.float32)
              + _b16(s1).astype(jnp.float32) * _b16(q1_s[:]).astype(jnp.float32))

    m = jnp.max(scores, axis=0, keepdims=True)
    ii = jax.lax.broadcasted_iota(jnp.int32, scores.shape, 0)
    li = jnp.min(jnp.where(scores == m, ii, BLOCK_S), axis=0, keepdims=True)
    onehot = (ii == li).astype(jnp.bfloat16)
    rows = jax.lax.dot_general(memb, onehot, (((0,), (0,)), ((), ())),
                               preferred_element_type=jnp.float32)

    upd = m > m_s[:]
    m_s[:] = jnp.where(upd, m, m_s[:])
    idx_s[:] = jnp.where(upd, li + step * BLOCK_S, idx_s[:])
    rows_s[:] = jnp.where(upd, rows, rows_s[:])

    @pl.when(step == nsteps - 1)
    def _fin():
        r_sel = jax.lax.dot_general(
            _b16(rows_s[:]), _b16(selt_ref[:]), (((1,), (0,)), ((), ())),
            preferred_element_type=jnp.float32)
        prod = _b16(wvt_ref[:]).astype(jnp.float32) * r_sel
        vals_ref[:] = jnp.sum(prod, axis=0, keepdims=True)
        bs_ref[:] = m_s[:]
        bi_ref[:] = idx_s[:]


def kernel(query_emb, memory_embs, WQ, bQ, WK, WV_small, WV_call):
    q2d = query_emb.reshape(1, D)
    WQ0, WQ1 = WQ[:, 0, :], WQ[:, 1, :]
    bq0, bq1 = bQ[:, 0].reshape(1, H), bQ[:, 1].reshape(1, H)
    WK0, WK1 = WK[:, 0, :], WK[:, 1, :]
    WVt = jnp.concatenate([WV_small[:, 0, :], WV_call], axis=0).T
    SelT = (jnp.arange(H)[:, None] ==
            jnp.minimum(jnp.arange(12)[None, :], 9)).astype(jnp.float32)

    nsteps = S // BLOCK_S
    full = lambda shape: pl.BlockSpec(shape, lambda i: (0, 0))
    vals, bs, bi = pl.pallas_call(
        _body,
        grid=(nsteps,),
        in_specs=[
            pl.BlockSpec((BLOCK_S, D), lambda i: (i, 0)),
            full((1, D)), full((H, D)), full((H, D)),
            full((1, H)), full((1, H)),
            full((H, D)), full((H, D)),
            full((D, 12)), full((H, 12)),
        ],
        out_specs=[full((1, 12)), full((1, H)), full((1, H))],
        out_shape=[
            jax.ShapeDtypeStruct((1, 12), jnp.float32),
            jax.ShapeDtypeStruct((1, H), jnp.float32),
            jax.ShapeDtypeStruct((1, H), jnp.int32),
        ],
        scratch_shapes=[
            pltpu.VMEM((1, H), jnp.float32),
            pltpu.VMEM((1, H), jnp.float32),
            pltpu.VMEM((1, H), jnp.float32),
            pltpu.VMEM((1, H), jnp.int32),
            pltpu.VMEM((D, H), jnp.float32),
        ],
    )(memory_embs, q2d, WQ0, WQ1, bq0, bq1, WK0, WK1, WVt, SelT)
    return vals.reshape(12), bs.reshape(10), bi.reshape(10)


# speedup vs baseline: 1.0363x; 1.0363x over previous
import jax
import jax.numpy as jnp
from jax.experimental import pallas as pl
from jax.experimental.pallas import tpu as pltpu

D = 768
S = 8192
H = 10
BLOCK_S = 2048


def _b16(x):
    return x.astype(jnp.bfloat16)


def _f32(x):
    return x.astype(jnp.float32)


def _body(mem_ref, q2d_ref, wqf_ref, bqf_ref, wkf_ref, wv12_ref,
          vals_ref, bs_ref, bi_ref,
          qp_s, m_s, idx_s, rows_s):
    step = pl.program_id(0)
    nsteps = pl.num_programs(0)

    @pl.when(step == 0)
    def _init():
        qcat = jax.lax.dot_general(
            _b16(q2d_ref[:]), _b16(wqf_ref[:]), (((1,), (1,)), ((), ())),
            preferred_element_type=jnp.float32) + bqf_ref[:]
        qp_s[:] = qcat
        m_s[:] = jnp.full((1, 2 * H), -jnp.inf, dtype=jnp.float32)
        idx_s[:] = jnp.zeros((1, 2 * H), dtype=jnp.int32)
        rows_s[:] = jnp.zeros((D, 2 * H), dtype=jnp.float32)

    memb = _b16(mem_ref[:])
    scat = jax.lax.dot_general(memb, _b16(wkf_ref[:]), (((1,), (1,)), ((), ())),
                               preferred_element_type=jnp.float32)
    p = _f32(_b16(scat)) * _f32(_b16(qp_s[:]))
    jj = jax.lax.broadcasted_iota(jnp.int32, p.shape, 1)
    even = (jj % 2) == 0
    scores = jnp.where(even, p + pltpu.roll(p, 2 * H - 1, 1), -jnp.inf)

    m = jnp.max(scores, axis=0, keepdims=True)
    ii = jax.lax.broadcasted_iota(jnp.int32, scores.shape, 0)
    li = jnp.min(jnp.where(scores == m, ii, BLOCK_S), axis=0, keepdims=True)
    onehot = (ii == li).astype(jnp.bfloat16)
    rows = jax.lax.dot_general(memb, onehot, (((0,), (0,)), ((), ())),
                               preferred_element_type=jnp.float32)

    upd = m > m_s[:]
    m_s[:] = jnp.where(upd, m, m_s[:])
    idx_s[:] = jnp.where(upd, li + step * BLOCK_S, idx_s[:])
    rows_s[:] = jnp.where(upd, rows, rows_s[:])

    @pl.when(step == nsteps - 1)
    def _fin():
        i20 = jax.lax.broadcasted_iota(jnp.int32, (2 * H, 12), 0)
        j12 = jax.lax.broadcasted_iota(jnp.int32, (2 * H, 12), 1)
        sel = (i20 == 2 * jnp.minimum(j12, 9)).astype(jnp.bfloat16)
        r_sel = jax.lax.dot_general(
            _b16(rows_s[:]), sel, (((1,), (0,)), ((), ())),
            preferred_element_type=jnp.float32)
        dd = jax.lax.dot_general(
            _b16(wv12_ref[:]), _b16(r_sel), (((1,), (0,)), ((), ())),
            preferred_element_type=jnp.float32)
        r12 = jax.lax.broadcasted_iota(jnp.int32, (12, 12), 0)
        c12 = jax.lax.broadcasted_iota(jnp.int32, (12, 12), 1)
        eye = (r12 == c12).astype(jnp.float32)
        vals_ref[:] = jnp.sum(dd * eye, axis=0, keepdims=True)
        bs_ref[:] = m_s[:]
        bi_ref[:] = idx_s[:]


def kernel(query_emb, memory_embs, WQ, bQ, WK, WV_small, WV_call):
    q2d = query_emb.reshape(1, D)
    WQf = WQ.reshape(2 * H, D)
    bQf = bQ.reshape(1, 2 * H)
    WKf = WK.reshape(2 * H, D)
    WV12 = jnp.concatenate([WV_small.reshape(9, D), WV_call], axis=0)

    nsteps = S // BLOCK_S
    full = lambda shape: pl.BlockSpec(shape, lambda i: (0, 0))
    vals, bs, bi = pl.pallas_call(
        _body,
        grid=(nsteps,),
        in_specs=[
            pl.BlockSpec((BLOCK_S, D), lambda i: (i, 0)),
            full((1, D)), full((2 * H, D)), full((1, 2 * H)),
            full((2 * H, D)), full((12, D)),
        ],
        out_specs=[full((1, 12)), full((1, 2 * H)), full((1, 2 * H))],
        out_shape=[
            jax.ShapeDtypeStruct((1, 12), jnp.float32),
            jax.ShapeDtypeStruct((1, 2 * H), jnp.float32),
            jax.ShapeDtypeStruct((1, 2 * H), jnp.int32),
        ],
        scratch_shapes=[
            pltpu.VMEM((1, 2 * H), jnp.float32),
            pltpu.VMEM((1, 2 * H), jnp.float32),
            pltpu.VMEM((1, 2 * H), jnp.int32),
            pltpu.VMEM((D, 2 * H), jnp.float32),
        ],
    )(memory_embs, q2d, WQf, bQf, WKf, WV12)
    return vals.reshape(12), bs[0, 0::2], bi[0, 0::2]


# speedup vs baseline: 1.1157x; 1.0766x over previous
import jax
import jax.numpy as jnp
from jax.experimental import pallas as pl
from jax.experimental.pallas import tpu as pltpu

D = 768
S = 8192
H = 10
W = 32
BLOCK_S = 2048

_SHIFTS = tuple(sorted({20 + j - 2 * min(j, 9) for j in range(12)}))
_DESTS = {s: tuple(j for j in range(12) if 20 + j - 2 * min(j, 9) == s)
          for s in _SHIFTS}


def _b16(x):
    return x.astype(jnp.bfloat16)


def _f32(x):
    return x.astype(jnp.float32)


def _head_to_val_lanes(x):
    lane = jax.lax.broadcasted_iota(jnp.int32, x.shape, 1)
    out = jnp.zeros(x.shape, dtype=jnp.float32)
    for s in _SHIFTS:
        rolled = pltpu.roll(x, s, 1)
        dmask = jnp.zeros(x.shape, dtype=jnp.float32)
        for j in _DESTS[s]:
            dmask = jnp.maximum(dmask, (lane == 20 + j).astype(jnp.float32))
        out = jnp.where(dmask > 0.5, rolled, out)
    return out


def _body(mem_ref, q2d_ref, wqf_ref, wall_ref, bqf_ref,
          vals_ref, bs_ref, bi_ref,
          qp_s, m_s, idx_s, v_s):
    step = pl.program_id(0)
    nsteps = pl.num_programs(0)

    @pl.when(step == 0)
    def _init():
        qcat = jax.lax.dot_general(
            _b16(q2d_ref[:]), _b16(wqf_ref[:]), (((1,), (1,)), ((), ())),
            preferred_element_type=jnp.float32) + bqf_ref[:]
        qp_s[:] = jnp.concatenate(
            [qcat, jnp.zeros((1, W - 2 * H), jnp.float32)], axis=1)
        m_s[:] = jnp.full((1, W), -jnp.inf, dtype=jnp.float32)
        idx_s[:] = jnp.zeros((1, W), dtype=jnp.int32)
        v_s[:] = jnp.zeros((1, W), dtype=jnp.float32)

    memb = _b16(mem_ref[:])
    scat = jax.lax.dot_general(memb, _b16(wall_ref[:]), (((1,), (1,)), ((), ())),
                               preferred_element_type=jnp.float32)
    p = _f32(_b16(scat)) * _f32(_b16(qp_s[:]))
    lane = jax.lax.broadcasted_iota(jnp.int32, p.shape, 1)
    score_lane = jnp.logical_and(lane % 2 == 0, lane < 2 * H)
    scores = jnp.where(score_lane, p + pltpu.roll(p, W - 1, 1), -jnp.inf)

    m = jnp.max(scores, axis=0, keepdims=True)
    ii = jax.lax.broadcasted_iota(jnp.int32, scores.shape, 0)
    li = jnp.min(jnp.where(scores == m, ii, BLOCK_S), axis=0, keepdims=True)
    onehot = (ii == li).astype(jnp.float32)
    maskv = _head_to_val_lanes(onehot)
    v_cand = jnp.sum(maskv * scat, axis=0, keepdims=True)

    upd = m > m_s[:]
    updv = _head_to_val_lanes(upd.astype(jnp.float32)) > 0.5
    m_s[:] = jnp.where(upd, m, m_s[:])
    idx_s[:] = jnp.where(upd, li + step * BLOCK_S, idx_s[:])
    v_s[:] = jnp.where(updv, v_cand, v_s[:])

    @pl.when(step == nsteps - 1)
    def _fin():
        vals_ref[:] = v_s[:]
        bs_ref[:] = m_s[:]
        bi_ref[:] = idx_s[:]


def kernel(query_emb, memory_embs, WQ, bQ, WK, WV_small, WV_call):
    q2d = query_emb.reshape(1, D)
    WALL = jnp.concatenate(
        [WK.reshape(2 * H, D), WV_small.reshape(9, D), WV_call], axis=0)
    WQf = WQ.reshape(2 * H, D)
    bQf = bQ.reshape(1, 2 * H)

    nsteps = S // BLOCK_S
    full = lambda shape: pl.BlockSpec(shape, lambda i: (0, 0))
    vals, bs, bi = pl.pallas_call(
        _body,
        grid=(nsteps,),
        in_specs=[
            pl.BlockSpec((BLOCK_S, D), lambda i: (i, 0)),
            full((1, D)), full((2 * H, D)), full((W, D)), full((1, 2 * H)),
        ],
        out_specs=[full((1, W)), full((1, W)), full((1, W))],
        out_shape=[
            jax.ShapeDtypeStruct((1, W), jnp.float32),
            jax.ShapeDtypeStruct((1, W), jnp.float32),
            jax.ShapeDtypeStruct((1, W), jnp.int32),
        ],
        scratch_shapes=[
            pltpu.VMEM((1, W), jnp.float32),
            pltpu.VMEM((1, W), jnp.float32),
            pltpu.VMEM((1, W), jnp.int32),
            pltpu.VMEM((1, W), jnp.float32),
        ],
    )(memory_embs, q2d, WQf, WALL, bQf)
    return vals[0, 2 * H:], bs[0, 0:2 * H:2], bi[0, 0:2 * H:2]


# speedup vs baseline: 1.3433x; 1.2039x over previous
import jax
import jax.numpy as jnp
from jax.experimental import pallas as pl
from jax.experimental.pallas import tpu as pltpu

D = 768
S = 8192
H = 10
W = 32
BLOCK_S = 2048

_SHIFTS = tuple(sorted({20 + j - 2 * min(j, 9) for j in range(12)}))
_DESTS = {s: tuple(j for j in range(12) if 20 + j - 2 * min(j, 9) == s)
          for s in _SHIFTS}


def _b16(x):
    return x.astype(jnp.bfloat16)


def _f32(x):
    return x.astype(jnp.float32)


def _head_to_val_lanes(x):
    lane = jax.lax.broadcasted_iota(jnp.int32, x.shape, 1)
    out = x
    for s in _SHIFTS:
        rolled = pltpu.roll(x, s, 1)
        dmask = jnp.zeros(x.shape, dtype=jnp.int32)
        for j in _DESTS[s]:
            dmask = jnp.maximum(dmask, (lane == 20 + j).astype(jnp.int32))
        out = jnp.where(dmask > 0, rolled, out)
    return out


def _body(mem_ref, q2d_ref, wqf_ref, wall_ref, bqf_ref,
          vals_ref, bs_ref, bi_ref,
          qp_s, m_s, idx_s, v_s):
    step = pl.program_id(0)
    nsteps = pl.num_programs(0)

    @pl.when(step == 0)
    def _init():
        qcat = jax.lax.dot_general(
            _b16(q2d_ref[:]), _b16(wqf_ref[:]), (((1,), (1,)), ((), ())),
            preferred_element_type=jnp.float32) + bqf_ref[:]
        qp_s[:] = jnp.concatenate(
            [qcat, jnp.zeros((1, W - 2 * H), jnp.float32)], axis=1)
        m_s[:] = jnp.full((1, W), -jnp.inf, dtype=jnp.float32)
        idx_s[:] = jnp.zeros((1, W), dtype=jnp.int32)
        v_s[:] = jnp.zeros((1, W), dtype=jnp.float32)

    memb = _b16(mem_ref[:])
    scat = jax.lax.dot_general(memb, _b16(wall_ref[:]), (((1,), (1,)), ((), ())),
                               preferred_element_type=jnp.float32)
    p = _f32(_b16(scat)) * _f32(_b16(qp_s[:]))
    lane = jax.lax.broadcasted_iota(jnp.int32, p.shape, 1)
    score_lane = jnp.logical_and(lane % 2 == 0, lane < 2 * H)
    scores = jnp.where(score_lane, p + pltpu.roll(p, W - 1, 1), -jnp.inf)

    m = jnp.max(scores, axis=0, keepdims=True)
    ii = jax.lax.broadcasted_iota(jnp.int32, scores.shape, 0)
    li = jnp.min(jnp.where(scores == m, ii, BLOCK_S), axis=0, keepdims=True)
    li_all = _head_to_val_lanes(li)
    sel = jnp.where(ii == li_all, scat, 0.0)
    v_cand = jnp.sum(sel, axis=0, keepdims=True)

    upd = m > m_s[:]
    updv = _head_to_val_lanes(upd.astype(jnp.int32)) > 0
    m_s[:] = jnp.where(upd, m, m_s[:])
    idx_s[:] = jnp.where(upd, li + step * BLOCK_S, idx_s[:])
    v_s[:] = jnp.where(updv, v_cand, v_s[:])

    @pl.when(step == nsteps - 1)
    def _fin():
        vals_ref[:] = v_s[:]
        bs_ref[:] = m_s[:]
        bi_ref[:] = idx_s[:]


def kernel(query_emb, memory_embs, WQ, bQ, WK, WV_small, WV_call):
    q2d = query_emb.reshape(1, D)
    WALL = jnp.concatenate(
        [WK.reshape(2 * H, D), WV_small.reshape(9, D), WV_call], axis=0)
    WQf = WQ.reshape(2 * H, D)
    bQf = bQ.reshape(1, 2 * H)

    nsteps = S // BLOCK_S
    full = lambda shape: pl.BlockSpec(shape, lambda i: (0, 0))
    vals, bs, bi = pl.pallas_call(
        _body,
        grid=(nsteps,),
        in_specs=[
            pl.BlockSpec((BLOCK_S, D), lambda i: (i, 0)),
            full((1, D)), full((2 * H, D)), full((W, D)), full((1, 2 * H)),
        ],
        out_specs=[full((1, W)), full((1, W)), full((1, W))],
        out_shape=[
            jax.ShapeDtypeStruct((1, W), jnp.float32),
            jax.ShapeDtypeStruct((1, W), jnp.float32),
            jax.ShapeDtypeStruct((1, W), jnp.int32),
        ],
        scratch_shapes=[
            pltpu.VMEM((1, W), jnp.float32),
            pltpu.VMEM((1, W), jnp.float32),
            pltpu.VMEM((1, W), jnp.int32),
            pltpu.VMEM((1, W), jnp.float32),
        ],
    )(memory_embs, q2d, WQf, WALL, bQf)
    return vals[0, 2 * H:], bs[0, 0:2 * H:2], bi[0, 0:2 * H:2]


# speedup vs baseline: 1.3481x; 1.0036x over previous
import jax
import jax.numpy as jnp
from jax.experimental import pallas as pl
from jax.experimental.pallas import tpu as pltpu

D = 768
S = 8192
H = 10
W = 32
BLOCK_S = 2048

_SHIFTS = tuple(sorted({20 + j - 2 * min(j, 9) for j in range(12)}))
_DESTS = {s: tuple(j for j in range(12) if 20 + j - 2 * min(j, 9) == s)
          for s in _SHIFTS}


def _b16(x):
    return x.astype(jnp.bfloat16)


def _f32(x):
    return x.astype(jnp.float32)


def _head_to_val_lanes(x):
    lane = jax.lax.broadcasted_iota(jnp.int32, x.shape, 1)
    out = x
    for s in _SHIFTS:
        rolled = pltpu.roll(x, s, 1)
        dmask = jnp.zeros(x.shape, dtype=jnp.int32)
        for j in _DESTS[s]:
            dmask = jnp.maximum(dmask, (lane == 20 + j).astype(jnp.int32))
        out = jnp.where(dmask > 0, rolled, out)
    return out


def _body(mem_ref, q2d_ref, wqf_ref, wall_ref, bqf_ref,
          vals_ref, bs_ref, bi_ref,
          qm_s, m_s, idx_s, v_s):
    step = pl.program_id(0)
    nsteps = pl.num_programs(0)

    @pl.when(step == 0)
    def _init():
        qrow = jax.lax.dot_general(
            _b16(q2d_ref[:]), _b16(wqf_ref[:]), (((1,), (1,)), ((), ())),
            preferred_element_type=jnp.float32) + bqf_ref[:]
        qrow32 = _f32(_b16(jnp.concatenate(
            [qrow, jnp.zeros((1, W - 2 * H), jnp.float32)], axis=1)))
        qfull = jnp.broadcast_to(qrow32, (W, W))
        rollc = pltpu.roll(qfull, W - 1, 1)
        rr = jax.lax.broadcasted_iota(jnp.int32, (W, W), 0)
        cc = jax.lax.broadcasted_iota(jnp.int32, (W, W), 1)
        head = rr < 2 * H
        even_diag = jnp.logical_and(jnp.logical_and(rr == cc, rr % 2 == 0), head)
        odd_sub = jnp.logical_and(jnp.logical_and(cc == rr - 1, rr % 2 == 1), head)
        qm_s[:] = _b16(jnp.where(even_diag, qfull, 0.0)
                       + jnp.where(odd_sub, rollc, 0.0))
        m_s[:] = jnp.full((1, W), -jnp.inf, dtype=jnp.float32)
        idx_s[:] = jnp.zeros((1, W), dtype=jnp.int32)
        v_s[:] = jnp.zeros((1, W), dtype=jnp.float32)

    memb = _b16(mem_ref[:])
    scat = jax.lax.dot_general(memb, _b16(wall_ref[:]), (((1,), (1,)), ((), ())),
                               preferred_element_type=jnp.float32)
    scores = jax.lax.dot_general(
        _b16(scat), qm_s[:], (((1,), (0,)), ((), ())),
        preferred_element_type=jnp.float32)

    m = jnp.max(scores, axis=0, keepdims=True)
    ii = jax.lax.broadcasted_iota(jnp.int32, scores.shape, 0)
    li = jnp.min(jnp.where(scores == m, ii, BLOCK_S), axis=0, keepdims=True)
    li_all = _head_to_val_lanes(li)
    sel = jnp.where(ii == li_all, scat, 0.0)
    v_cand = jnp.sum(sel, axis=0, keepdims=True)

    upd = m > m_s[:]
    updv = _head_to_val_lanes(upd.astype(jnp.int32)) > 0
    m_s[:] = jnp.where(upd, m, m_s[:])
    idx_s[:] = jnp.where(upd, li + step * BLOCK_S, idx_s[:])
    v_s[:] = jnp.where(updv, v_cand, v_s[:])

    @pl.when(step == nsteps - 1)
    def _fin():
        vals_ref[:] = v_s[:]
        bs_ref[:] = m_s[:]
        bi_ref[:] = idx_s[:]


def kernel(query_emb, memory_embs, WQ, bQ, WK, WV_small, WV_call):
    q2d = query_emb.reshape(1, D)
    WALL = jnp.concatenate(
        [WK.reshape(2 * H, D), WV_small.reshape(9, D), WV_call], axis=0)
    WQf = WQ.reshape(2 * H, D)
    bQf = bQ.reshape(1, 2 * H)

    nsteps = S // BLOCK_S
    full = lambda shape: pl.BlockSpec(shape, lambda i: (0, 0))
    vals, bs, bi = pl.pallas_call(
        _body,
        grid=(nsteps,),
        in_specs=[
            pl.BlockSpec((BLOCK_S, D), lambda i: (i, 0)),
            full((1, D)), full((2 * H, D)), full((W, D)), full((1, 2 * H)),
        ],
        out_specs=[full((1, W)), full((1, W)), full((1, W))],
        out_shape=[
            jax.ShapeDtypeStruct((1, W), jnp.float32),
            jax.ShapeDtypeStruct((1, W), jnp.float32),
            jax.ShapeDtypeStruct((1, W), jnp.int32),
        ],
        scratch_shapes=[
            pltpu.VMEM((W, W), jnp.bfloat16),
            pltpu.VMEM((1, W), jnp.float32),
            pltpu.VMEM((1, W), jnp.int32),
            pltpu.VMEM((1, W), jnp.float32),
        ],
    )(memory_embs, q2d, WQf, WALL, bQf)
    return vals[0, 2 * H:], bs[0, 0:2 * H:2], bi[0, 0:2 * H:2]


# speedup vs baseline: 1.3563x; 1.0061x over previous
import jax
import jax.numpy as jnp
from jax.experimental import pallas as pl
from jax.experimental.pallas import tpu as pltpu

D = 768
S = 8192
H = 10
W = 32
BLOCK_S = 2048

_SHIFTS = tuple(sorted({20 + j - 2 * min(j, 9) for j in range(12)}))
_DESTS = {s: tuple(j for j in range(12) if 20 + j - 2 * min(j, 9) == s)
          for s in _SHIFTS}


def _b16(x):
    return x.astype(jnp.bfloat16)


def _f32(x):
    return x.astype(jnp.float32)


def _head_to_val_lanes(x):
    lane = jax.lax.broadcasted_iota(jnp.int32, x.shape, 1)
    out = x
    for s in _SHIFTS:
        rolled = pltpu.roll(x, s, 1)
        dmask = jnp.zeros(x.shape, dtype=jnp.int32)
        for j in _DESTS[s]:
            dmask = jnp.maximum(dmask, (lane == 20 + j).astype(jnp.int32))
        out = jnp.where(dmask > 0, rolled, out)
    return out


def _body(mem_ref, q2d_ref, wqf_ref, wall_ref, bqf_ref,
          vals_ref, bs_ref, bi_ref,
          qm_s, m_s, idx_s, v_s):
    step = pl.program_id(0)
    nsteps = pl.num_programs(0)

    @pl.when(step == 0)
    def _init():
        qrow = jax.lax.dot_general(
            _b16(q2d_ref[:]), _b16(wqf_ref[:]), (((1,), (1,)), ((), ())),
            preferred_element_type=jnp.float32) + bqf_ref[:]
        qrow32 = _f32(_b16(jnp.concatenate(
            [qrow, jnp.zeros((1, W - 2 * H), jnp.float32)], axis=1)))
        qfull = jnp.broadcast_to(qrow32, (W, W))
        rollc = pltpu.roll(qfull, W - 1, 1)
        rr = jax.lax.broadcasted_iota(jnp.int32, (W, W), 0)
        cc = jax.lax.broadcasted_iota(jnp.int32, (W, W), 1)
        head = rr < 2 * H
        even_diag = jnp.logical_and(jnp.logical_and(rr == cc, rr % 2 == 0), head)
        odd_sub = jnp.logical_and(jnp.logical_and(cc == rr - 1, rr % 2 == 1), head)
        qm_s[:] = _b16(jnp.where(even_diag, qfull, 0.0)
                       + jnp.where(odd_sub, rollc, 0.0))
        m_s[:] = jnp.full((1, W), -jnp.inf, dtype=jnp.float32)
        idx_s[:] = jnp.zeros((1, W), dtype=jnp.int32)
        v_s[:] = jnp.zeros((1, W), dtype=jnp.float32)

    scat = jax.lax.dot_general(mem_ref[:], wall_ref[:], (((1,), (1,)), ((), ())),
                               preferred_element_type=jnp.float32)
    scores = jax.lax.dot_general(
        scat, _f32(qm_s[:]), (((1,), (0,)), ((), ())),
        preferred_element_type=jnp.float32)

    m = jnp.max(scores, axis=0, keepdims=True)
    ii = jax.lax.broadcasted_iota(jnp.int32, scores.shape, 0)
    li = jnp.min(jnp.where(scores == m, ii, BLOCK_S), axis=0, keepdims=True)
    li_all = _head_to_val_lanes(li)
    sel = jnp.where(ii == li_all, scat, 0.0)
    v_cand = jnp.sum(sel, axis=0, keepdims=True)

    upd = m > m_s[:]
    updv = _head_to_val_lanes(upd.astype(jnp.int32)) > 0
    m_s[:] = jnp.where(upd, m, m_s[:])
    idx_s[:] = jnp.where(upd, li + step * BLOCK_S, idx_s[:])
    v_s[:] = jnp.where(updv, v_cand, v_s[:])

    @pl.when(step == nsteps - 1)
    def _fin():
        vals_ref[:] = v_s[:]
        bs_ref[:] = m_s[:]
        bi_ref[:] = idx_s[:]


def kernel(query_emb, memory_embs, WQ, bQ, WK, WV_small, WV_call):
    q2d = query_emb.reshape(1, D)
    WALL = jnp.concatenate(
        [WK.reshape(2 * H, D), WV_small.reshape(9, D), WV_call], axis=0)
    WQf = WQ.reshape(2 * H, D)
    bQf = bQ.reshape(1, 2 * H)

    nsteps = S // BLOCK_S
    full = lambda shape: pl.BlockSpec(shape, lambda i: (0, 0))
    vals, bs, bi = pl.pallas_call(
        _body,
        grid=(nsteps,),
        in_specs=[
            pl.BlockSpec((BLOCK_S, D), lambda i: (i, 0)),
            full((1, D)), full((2 * H, D)), full((W, D)), full((1, 2 * H)),
        ],
        out_specs=[full((1, W)), full((1, W)), full((1, W))],
        out_shape=[
            jax.ShapeDtypeStruct((1, W), jnp.float32),
            jax.ShapeDtypeStruct((1, W), jnp.float32),
            jax.ShapeDtypeStruct((1, W), jnp.int32),
        ],
        scratch_shapes=[
            pltpu.VMEM((W, W), jnp.bfloat16),
            pltpu.VMEM((1, W), jnp.float32),
            pltpu.VMEM((1, W), jnp.int32),
            pltpu.VMEM((1, W), jnp.float32),
        ],
    )(memory_embs, q2d, WQf, WALL, bQf)
    return vals[0, 2 * H:], bs[0, 0:2 * H:2], bi[0, 0:2 * H:2]
